# baseline (speedup 1.0000x reference)
import functools

import jax
import jax.numpy as jnp
from jax import lax
from jax.experimental import pallas as pl
from jax.experimental.pallas import tpu as pltpu
from jax.experimental.pallas import tpu_sc as plsc

DIM = 128
NV = 50000
NL = 100000
NCL = 150000
NE = 600000
NE_PAD = NE + 512
B = 2000

N_ROUNDS = 4
NSC = 2
NSUB = 16


def _mlp_math(x, w1t, b1, w2t, b2):
    h = jnp.maximum(jnp.dot(x.astype(jnp.bfloat16), w1t,
                            preferred_element_type=jnp.float32) + b1, 0.0)
    return jnp.dot(h.astype(jnp.bfloat16), w2t,
                   preferred_element_type=jnp.float32) + b2


def _mlp_body(x_ref, w1t_ref, b1_ref, w2t_ref, b2_ref, o_ref):
    o_ref[...] = _mlp_math(x_ref[...], w1t_ref[...], b1_ref[...],
                           w2t_ref[...], b2_ref[...])


def _mlp_call(x, w1t, b1, w2t, b2):
    n = x.shape[0]
    assert n % B == 0
    return pl.pallas_call(
        _mlp_body,
        grid=(n // B,),
        in_specs=[
            pl.BlockSpec((B, DIM), lambda i: (i, 0)),
            pl.BlockSpec((DIM, DIM), lambda i: (0, 0)),
            pl.BlockSpec((1, DIM), lambda i: (0, 0)),
            pl.BlockSpec((DIM, DIM), lambda i: (0, 0)),
            pl.BlockSpec((1, DIM), lambda i: (0, 0)),
        ],
        out_specs=pl.BlockSpec((B, DIM), lambda i: (i, 0)),
        out_shape=jax.ShapeDtypeStruct((n, DIM), jnp.float32),
    )(x, w1t, b1, w2t, b2)


def _lstm_gates(g, c):
    i = g[:, :DIM]
    f = g[:, DIM:2 * DIM]
    gg = g[:, 2 * DIM:3 * DIM]
    o = g[:, 3 * DIM:]
    c_new = jax.nn.sigmoid(f) * c + jax.nn.sigmoid(i) * jnp.tanh(gg)
    h_new = jax.nn.sigmoid(o) * jnp.tanh(c_new)
    return h_new, c_new


def _clstm_body(x_ref, h_ref, c_ref, wih_ref, whh_ref, b_ref,
                w1t_ref, b1_ref, w2t_ref, b2_ref,
                h_out, c_out, pre_out):
    g = (jnp.dot(x_ref[...].astype(jnp.bfloat16), wih_ref[...],
                 preferred_element_type=jnp.float32)
         + jnp.dot(h_ref[...].astype(jnp.bfloat16), whh_ref[...],
                   preferred_element_type=jnp.float32)
         + b_ref[...])
    h_new, c_new = _lstm_gates(g, c_ref[...])
    h_out[...] = h_new
    c_out[...] = c_new
    pre_out[...] = _mlp_math(h_new, w1t_ref[...], b1_ref[...],
                             w2t_ref[...], b2_ref[...])


def _clstm_call(x_pad, h, c, wih_t, whh_t, b, w1t, b1, w2t, b2):
    n = h.shape[0]
    assert n % B == 0
    row = pl.BlockSpec((B, DIM), lambda i: (i, 0))
    full = lambda r, cdim: pl.BlockSpec((r, cdim), lambda i: (0, 0))
    return pl.pallas_call(
        _clstm_body,
        grid=(n // B,),
        in_specs=[
            row, row, row,
            full(DIM, 4 * DIM), full(DIM, 4 * DIM), full(1, 4 * DIM),
            full(DIM, DIM), full(1, DIM), full(DIM, DIM), full(1, DIM),
        ],
        out_specs=[row, row, row],
        out_shape=[
            jax.ShapeDtypeStruct((n, DIM), jnp.float32),
            jax.ShapeDtypeStruct((n, DIM), jnp.float32),
            jax.ShapeDtypeStruct((n, DIM), jnp.float32),
        ],
    )(x_pad, h, c, wih_t, whh_t, b, w1t, b1, w2t, b2)


def _llstm_body(cl_ref, flip_ref, h_ref, c_ref,
                wih_a_ref, wih_b_ref, whh_ref, b_ref,
                h_out, c_out):
    g = (jnp.dot(cl_ref[...].astype(jnp.bfloat16), wih_a_ref[...],
                 preferred_element_type=jnp.float32)
         + jnp.dot(flip_ref[...].astype(jnp.bfloat16), wih_b_ref[...],
                   preferred_element_type=jnp.float32)
         + jnp.dot(h_ref[...].astype(jnp.bfloat16), whh_ref[...],
                   preferred_element_type=jnp.float32)
         + b_ref[...])
    h_new, c_new = _lstm_gates(g, c_ref[...])
    h_out[...] = h_new
    c_out[...] = c_new


def _llstm_call(cl_pad, h, c, wih_a_t, wih_b_t, whh_t, b):
    n = h.shape[0]
    assert n % B == 0
    nb = n // B
    half = nb // 2
    row = pl.BlockSpec((B, DIM), lambda i: (i, 0))
    flip_spec = pl.BlockSpec((B, DIM), lambda i: ((i + half) % nb, 0))
    full = lambda r, cdim: pl.BlockSpec((r, cdim), lambda i: (0, 0))
    return pl.pallas_call(
        _llstm_body,
        grid=(nb,),
        in_specs=[
            row, flip_spec, row, row,
            full(DIM, 4 * DIM), full(DIM, 4 * DIM), full(DIM, 4 * DIM),
            full(1, 4 * DIM),
        ],
        out_specs=[row, row],
        out_shape=[
            jax.ShapeDtypeStruct((n, DIM), jnp.float32),
            jax.ShapeDtypeStruct((n, DIM), jnp.float32),
        ],
    )(cl_pad, h, h, c, wih_a_t, wih_b_t, whh_t, b)


def _vote_body(a_ref, b2h_ref, w1a_ref, w1b_ref, b1_ref, w2t_ref, b2_ref, o_ref):
    h = jnp.maximum(
        jnp.dot(a_ref[...], w1a_ref[...], preferred_element_type=jnp.float32)
        + jnp.dot(b2h_ref[...], w1b_ref[...], preferred_element_type=jnp.float32)
        + b1_ref[...], 0.0)
    o_ref[...] = jnp.dot(h, w2t_ref[...], preferred_element_type=jnp.float32) + b2_ref[...]


def _vote_call(l_h, w1a_t, w1b_t, b1, w2t, b2):
    nb = NV // B
    half = NL // B // 2
    row = pl.BlockSpec((B, DIM), lambda i: (i, 0))
    row_b = pl.BlockSpec((B, DIM), lambda i: (i + half, 0))
    full = lambda r, cdim: pl.BlockSpec((r, cdim), lambda i: (0, 0))
    return pl.pallas_call(
        _vote_body,
        grid=(nb,),
        in_specs=[
            row, row_b,
            full(DIM, DIM), full(DIM, DIM), full(1, DIM),
            full(DIM, 1), full(1, 1),
        ],
        out_specs=pl.BlockSpec((B, 1), lambda i: (i, 0)),
        out_shape=jax.ShapeDtypeStruct((NV, 1), jnp.float32),
    )(l_h, l_h, w1a_t, w1b_t, b1, w2t, b2)


def _lane(v, i, iota):
    return jnp.sum(jnp.where(iota == i, v, 0), axis=0)


def _make_segsum(n_in, n_pass, r, nbounds_pad):
    assert r % 8 == 0
    stride = r + 8
    n_out_pad = n_pass * NSC * NSUB * r
    mesh = plsc.VectorSubcoreMesh(core_axis_name="c", subcore_axis_name="s")

    @functools.partial(
        pl.kernel,
        out_type=jax.ShapeDtypeStruct((n_out_pad, DIM), jnp.float32),
        mesh=mesh,
        compiler_params=pltpu.CompilerParams(needs_layout_passes=False),
        scratch_types=[
            pltpu.VMEM((nbounds_pad,), jnp.int32),
            pltpu.VMEM((2, 128), jnp.int32),
            pltpu.VMEM((2, 128), jnp.int32),
            pltpu.VMEM((2, 128, DIM), jnp.float32),
            pltpu.VMEM((32, DIM), jnp.float32),
            pltpu.VMEM_SHARED((NSUB * stride, DIM), jnp.float32),
            pltpu.SemaphoreType.DMA((2,)),
            pltpu.SemaphoreType.DMA((2,)),
            pltpu.SemaphoreType.DMA((2,)),
            pltpu.SemaphoreType.DMA,
        ],
    )
    def seg(g_hbm, l_hbm, b_hbm, x_hbm, out_hbm,
            bounds_v, gi_v, li_v, rows_v, zeros_v, acc,
            sem_g, sem_i, sem_l, sem_z):
        c = lax.axis_index("c")
        s = lax.axis_index("s")
        iota = lax.iota(jnp.int32, 16)
        pltpu.sync_copy(b_hbm, bounds_v)

        def zloop(i, carry):
            for j in range(8):
                zeros_v[i, pl.ds(j * 16, 16)] = jnp.zeros((16,), jnp.float32)
            return carry
        lax.fori_loop(0, 32, zloop, 0)

        base_row = s * stride

        def idx_issue(b, woff):
            return (pltpu.async_copy(g_hbm.at[pl.ds(woff, 128)],
                                     gi_v.at[b], sem_i.at[b]),
                    pltpu.async_copy(l_hbm.at[pl.ds(woff, 128)],
                                     li_v.at[b], sem_l.at[b]))

        def gather_issue(b):
            return pltpu.async_copy(x_hbm.at[gi_v.at[b]], rows_v.at[b],
                                    sem_g.at[b])

        def gather_wait(b):
            pltpu.make_async_copy(x_hbm.at[gi_v.at[b]], rows_v.at[b],
                                  sem_g.at[b]).wait()

        def scatter(b):
            pltpu.sync_copy(rows_v.at[b], acc.at[li_v.at[b]], add=True)

        for p in range(n_pass):
            slot = (p * NSC + c) * NSUB + s
            zd = []
            off = 0
            while off < stride:
                sz = min(32, stride - off)
                zd.append(pltpu.async_copy(
                    zeros_v.at[pl.ds(0, sz)],
                    acc.at[pl.ds(base_row + off, sz)], sem_z))
                off += sz
            bw = bounds_v[pl.ds(slot, 16)]
            start = _lane(bw, 0, iota)
            end = _lane(bw, 1, iota)
            wbase = pl.multiple_of(start & ~7, 8)
            npair = (end - wbase + 255) // 256

            def mask_fix(b, woff):
                for j in range(8):
                    pos = woff + j * 16 + iota
                    li = li_v[b, pl.ds(j * 16, 16)]
                    valid = (pos >= start) & (pos < end)
                    li_v[b, pl.ds(j * 16, 16)] = (
                        jnp.where(valid, li, r) + base_row)

            i0, l0 = idx_issue(0, wbase)
            i0.wait()
            l0.wait()
            gather_issue(0)
            for d in zd:
                d.wait()

            def pair(j, carry):
                woff0 = pl.multiple_of(wbase + j * 256, 8)
                woff1 = woff0 + 128
                woff2 = woff0 + 256
                i1, l1 = idx_issue(1, woff1)
                mask_fix(0, woff0)
                gather_wait(0)
                i1.wait()
                l1.wait()
                g1 = gather_issue(1)
                scatter(0)
                i2, l2 = idx_issue(0, woff2)
                mask_fix(1, woff1)
                g1.wait()
                i2.wait()
                l2.wait()
                gather_issue(0)
                scatter(1)
                return carry
            lax.fori_loop(0, npair, pair, 0)
            gather_wait(0)
            pltpu.sync_copy(acc.at[pl.ds(base_row, r)],
                            out_hbm.at[pl.ds(slot * r, r)])

    return seg


_SEG_LC = _make_segsum(NL, 7, 720, 256)
_SEG_CL = _make_segsum(NCL, 5, 720, 192)


def _sorted_cells(dst, src, r, nslots, nbounds_pad):
    order = jnp.argsort(dst)
    dst_s = dst[order].astype(jnp.int32)
    src_s = src[order].astype(jnp.int32)
    loc = (dst_s % r).astype(jnp.int32)
    bounds = jnp.searchsorted(
        dst_s, jnp.arange(nslots + 1, dtype=jnp.int32) * r).astype(jnp.int32)
    bounds = jnp.concatenate(
        [bounds, jnp.full((nbounds_pad - nslots - 1,), NE, jnp.int32)])
    src_pad = jnp.concatenate([src_s, jnp.zeros((NE_PAD - NE,), jnp.int32)])
    loc_pad = jnp.concatenate([loc, jnp.full((NE_PAD - NE,), r, jnp.int32)])
    return src_pad, loc_pad, bounds


def kernel(lit_idx, clause_idx, n_vars, n_lits, n_clauses,
           L_init_w, L_init_b, C_init_w, C_init_b,
           Lmsg_W1, Lmsg_b1, Lmsg_W2, Lmsg_b2,
           Cmsg_W1, Cmsg_b1, Cmsg_W2, Cmsg_b2,
           Lu_Wih, Lu_Whh, Lu_bih, Lu_bhh,
           Cu_Wih, Cu_Whh, Cu_bih, Cu_bhh,
           V_W1, V_b1, V_W2, V_b2):
    lit_idx = lit_idx.astype(jnp.int32)
    clause_idx = clause_idx.astype(jnp.int32)

    lc_src, lc_loc, lc_bounds = _sorted_cells(clause_idx, lit_idx, 720, 224, 256)
    cl_src, cl_loc, cl_bounds = _sorted_cells(lit_idx, clause_idx, 720, 160, 192)

    bf = jnp.bfloat16
    lmsg_w1t = Lmsg_W1.T.astype(bf)
    lmsg_w2t = Lmsg_W2.T.astype(bf)
    cmsg_w1t = Cmsg_W1.T.astype(bf)
    cmsg_w2t = Cmsg_W2.T.astype(bf)
    lmsg_b1 = Lmsg_b1.reshape(1, DIM)
    lmsg_b2 = Lmsg_b2.reshape(1, DIM)
    cmsg_b1 = Cmsg_b1.reshape(1, DIM)
    cmsg_b2 = Cmsg_b2.reshape(1, DIM)
    cu_wih_t = Cu_Wih.T.astype(bf)
    cu_whh_t = Cu_Whh.T.astype(bf)
    cu_b = (Cu_bih + Cu_bhh).reshape(1, 4 * DIM)
    lu_wih_a_t = Lu_Wih.T[:DIM].astype(bf)
    lu_wih_b_t = Lu_Wih.T[DIM:].astype(bf)
    lu_whh_t = Lu_Whh.T.astype(bf)
    lu_b = (Lu_bih + Lu_bhh).reshape(1, 4 * DIM)
    v_w1a_t = V_W1.T[:DIM]
    v_w1b_t = V_W1.T[DIM:]
    v_b1 = V_b1.reshape(1, DIM)
    v_w2t = V_W2.T
    v_b2 = V_b2.reshape(1, 1)

    l0 = (L_init_w[:, 0] + L_init_b).reshape(1, DIM)
    c0 = (C_init_w[:, 0] + C_init_b).reshape(1, DIM)
    L_h = jnp.tile(l0, (NL, 1))
    L_c = jnp.zeros((NL, DIM), jnp.float32)
    C_h = jnp.tile(c0, (NCL, 1))
    C_c = jnp.zeros((NCL, DIM), jnp.float32)

    for _ in range(N_ROUNDS):
        L_pre = _mlp_call(L_h, lmsg_w1t, lmsg_b1, lmsg_w2t, lmsg_b2)
        LC = _SEG_LC(lc_src, lc_loc, lc_bounds, L_pre)
        C_h, C_c, C_pre = _clstm_call(LC, C_h, C_c, cu_wih_t, cu_whh_t, cu_b,
                                      cmsg_w1t, cmsg_b1, cmsg_w2t, cmsg_b2)
        CL = _SEG_CL(cl_src, cl_loc, cl_bounds, C_pre)
        L_h, L_c = _llstm_call(CL, L_h, L_c, lu_wih_a_t, lu_wih_b_t,
                               lu_whh_t, lu_b)

    return _vote_call(L_h, v_w1a_t, v_w1b_t, v_b1, v_w2t, v_b2)


# speedup vs baseline: 1.0751x; 1.0751x over previous
import functools

import jax
import jax.numpy as jnp
from jax import lax
from jax.experimental import pallas as pl
from jax.experimental.pallas import tpu as pltpu
from jax.experimental.pallas import tpu_sc as plsc

DIM = 128
NV = 50000
NL = 100000
NCL = 150000
NE = 600000
NE_PAD = NE + 512
B = 2000

N_ROUNDS = 4
NSC = 2
NSUB = 16


def _mlp_math(x, w1t, b1, w2t, b2):
    h = jnp.maximum(jnp.dot(x.astype(jnp.bfloat16), w1t,
                            preferred_element_type=jnp.float32) + b1, 0.0)
    return jnp.dot(h.astype(jnp.bfloat16), w2t,
                   preferred_element_type=jnp.float32) + b2


def _mlp_body(x_ref, w1t_ref, b1_ref, w2t_ref, b2_ref, o_ref):
    o_ref[...] = _mlp_math(x_ref[...], w1t_ref[...], b1_ref[...],
                           w2t_ref[...], b2_ref[...])


def _mlp_call(x, w1t, b1, w2t, b2):
    n = x.shape[0]
    assert n % B == 0
    return pl.pallas_call(
        _mlp_body,
        grid=(n // B,),
        in_specs=[
            pl.BlockSpec((B, DIM), lambda i: (i, 0)),
            pl.BlockSpec((DIM, DIM), lambda i: (0, 0)),
            pl.BlockSpec((1, DIM), lambda i: (0, 0)),
            pl.BlockSpec((DIM, DIM), lambda i: (0, 0)),
            pl.BlockSpec((1, DIM), lambda i: (0, 0)),
        ],
        out_specs=pl.BlockSpec((B, DIM), lambda i: (i, 0)),
        out_shape=jax.ShapeDtypeStruct((n, DIM), jnp.float32),
    )(x, w1t, b1, w2t, b2)


def _lstm_gates(g, c):
    i = g[:, :DIM]
    f = g[:, DIM:2 * DIM]
    gg = g[:, 2 * DIM:3 * DIM]
    o = g[:, 3 * DIM:]
    c_new = jax.nn.sigmoid(f) * c + jax.nn.sigmoid(i) * jnp.tanh(gg)
    h_new = jax.nn.sigmoid(o) * jnp.tanh(c_new)
    return h_new, c_new


def _clstm_body(x_ref, h_ref, c_ref, wih_ref, whh_ref, b_ref,
                w1t_ref, b1_ref, w2t_ref, b2_ref,
                h_out, c_out, pre_out):
    g = (jnp.dot(x_ref[...].astype(jnp.bfloat16), wih_ref[...],
                 preferred_element_type=jnp.float32)
         + jnp.dot(h_ref[...].astype(jnp.bfloat16), whh_ref[...],
                   preferred_element_type=jnp.float32)
         + b_ref[...])
    h_new, c_new = _lstm_gates(g, c_ref[...])
    h_out[...] = h_new
    c_out[...] = c_new
    pre_out[...] = _mlp_math(h_new, w1t_ref[...], b1_ref[...],
                             w2t_ref[...], b2_ref[...])


def _clstm_call(x_pad, h, c, wih_t, whh_t, b, w1t, b1, w2t, b2):
    n = h.shape[0]
    assert n % B == 0
    row = pl.BlockSpec((B, DIM), lambda i: (i, 0))
    full = lambda r, cdim: pl.BlockSpec((r, cdim), lambda i: (0, 0))
    return pl.pallas_call(
        _clstm_body,
        grid=(n // B,),
        in_specs=[
            row, row, row,
            full(DIM, 4 * DIM), full(DIM, 4 * DIM), full(1, 4 * DIM),
            full(DIM, DIM), full(1, DIM), full(DIM, DIM), full(1, DIM),
        ],
        out_specs=[row, row, row],
        out_shape=[
            jax.ShapeDtypeStruct((n, DIM), jnp.float32),
            jax.ShapeDtypeStruct((n, DIM), jnp.float32),
            jax.ShapeDtypeStruct((n, DIM), jnp.float32),
        ],
    )(x_pad, h, c, wih_t, whh_t, b, w1t, b1, w2t, b2)


def _llstm_body(cl_ref, flip_ref, h_ref, c_ref,
                wih_a_ref, wih_b_ref, whh_ref, b_ref,
                h_out, c_out):
    g = (jnp.dot(cl_ref[...].astype(jnp.bfloat16), wih_a_ref[...],
                 preferred_element_type=jnp.float32)
         + jnp.dot(flip_ref[...].astype(jnp.bfloat16), wih_b_ref[...],
                   preferred_element_type=jnp.float32)
         + jnp.dot(h_ref[...].astype(jnp.bfloat16), whh_ref[...],
                   preferred_element_type=jnp.float32)
         + b_ref[...])
    h_new, c_new = _lstm_gates(g, c_ref[...])
    h_out[...] = h_new
    c_out[...] = c_new


def _llstm_call(cl_pad, h, c, wih_a_t, wih_b_t, whh_t, b):
    n = h.shape[0]
    assert n % B == 0
    nb = n // B
    half = nb // 2
    row = pl.BlockSpec((B, DIM), lambda i: (i, 0))
    flip_spec = pl.BlockSpec((B, DIM), lambda i: ((i + half) % nb, 0))
    full = lambda r, cdim: pl.BlockSpec((r, cdim), lambda i: (0, 0))
    return pl.pallas_call(
        _llstm_body,
        grid=(nb,),
        in_specs=[
            row, flip_spec, row, row,
            full(DIM, 4 * DIM), full(DIM, 4 * DIM), full(DIM, 4 * DIM),
            full(1, 4 * DIM),
        ],
        out_specs=[row, row],
        out_shape=[
            jax.ShapeDtypeStruct((n, DIM), jnp.float32),
            jax.ShapeDtypeStruct((n, DIM), jnp.float32),
        ],
    )(cl_pad, h, h, c, wih_a_t, wih_b_t, whh_t, b)


def _vote_body(a_ref, b2h_ref, w1a_ref, w1b_ref, b1_ref, w2t_ref, b2_ref, o_ref):
    h = jnp.maximum(
        jnp.dot(a_ref[...], w1a_ref[...], preferred_element_type=jnp.float32)
        + jnp.dot(b2h_ref[...], w1b_ref[...], preferred_element_type=jnp.float32)
        + b1_ref[...], 0.0)
    o_ref[...] = jnp.dot(h, w2t_ref[...], preferred_element_type=jnp.float32) + b2_ref[...]


def _vote_call(l_h, w1a_t, w1b_t, b1, w2t, b2):
    nb = NV // B
    half = NL // B // 2
    row = pl.BlockSpec((B, DIM), lambda i: (i, 0))
    row_b = pl.BlockSpec((B, DIM), lambda i: (i + half, 0))
    full = lambda r, cdim: pl.BlockSpec((r, cdim), lambda i: (0, 0))
    return pl.pallas_call(
        _vote_body,
        grid=(nb,),
        in_specs=[
            row, row_b,
            full(DIM, DIM), full(DIM, DIM), full(1, DIM),
            full(DIM, 1), full(1, 1),
        ],
        out_specs=pl.BlockSpec((B, 1), lambda i: (i, 0)),
        out_shape=jax.ShapeDtypeStruct((NV, 1), jnp.float32),
    )(l_h, l_h, w1a_t, w1b_t, b1, w2t, b2)


def _lane(v, i, iota):
    return jnp.sum(jnp.where(iota == i, v, 0), axis=0)


def _make_segsum(n_in, n_pass, r, nbounds_pad):
    assert r % 8 == 0
    stride = r + 8
    n_out_pad = n_pass * NSC * NSUB * r
    mesh = plsc.VectorSubcoreMesh(core_axis_name="c", subcore_axis_name="s")

    @functools.partial(
        pl.kernel,
        out_type=jax.ShapeDtypeStruct((n_out_pad, DIM), jnp.float32),
        mesh=mesh,
        compiler_params=pltpu.CompilerParams(needs_layout_passes=False),
        scratch_types=[
            pltpu.VMEM((nbounds_pad,), jnp.int32),
            pltpu.VMEM((2, 128), jnp.int32),
            pltpu.VMEM((2, 128), jnp.int32),
            pltpu.VMEM((2, 128, DIM), jnp.float32),
            pltpu.VMEM((32, DIM), jnp.float32),
            pltpu.VMEM_SHARED((NSUB * stride, DIM), jnp.float32),
            pltpu.SemaphoreType.DMA((2,)),
            pltpu.SemaphoreType.DMA((2,)),
            pltpu.SemaphoreType.DMA((2,)),
            pltpu.SemaphoreType.DMA,
        ],
    )
    def seg(g_hbm, l_hbm, b_hbm, x_hbm, out_hbm,
            bounds_v, gi_v, li_v, rows_v, zeros_v, acc,
            sem_g, sem_i, sem_l, sem_z):
        c = lax.axis_index("c")
        s = lax.axis_index("s")
        iota = lax.iota(jnp.int32, 16)
        pltpu.sync_copy(b_hbm, bounds_v)

        def zloop(i, carry):
            for j in range(8):
                zeros_v[i, pl.ds(j * 16, 16)] = jnp.zeros((16,), jnp.float32)
            return carry
        lax.fori_loop(0, 32, zloop, 0)

        base_row = s * stride

        def idx_issue(b, woff):
            return (pltpu.async_copy(g_hbm.at[pl.ds(woff, 128)],
                                     gi_v.at[b], sem_i.at[b]),
                    pltpu.async_copy(l_hbm.at[pl.ds(woff, 128)],
                                     li_v.at[b], sem_l.at[b]))

        def gather_issue(b):
            return pltpu.async_copy(x_hbm.at[gi_v.at[b]], rows_v.at[b],
                                    sem_g.at[b])

        def gather_wait(b):
            pltpu.make_async_copy(x_hbm.at[gi_v.at[b]], rows_v.at[b],
                                  sem_g.at[b]).wait()

        def scatter(b):
            pltpu.sync_copy(rows_v.at[b], acc.at[li_v.at[b]], add=True)

        for p in range(n_pass):
            slot = (p * NSC + c) * NSUB + s
            zd = []
            off = 0
            while off < stride:
                sz = min(32, stride - off)
                zd.append(pltpu.async_copy(
                    zeros_v.at[pl.ds(0, sz)],
                    acc.at[pl.ds(base_row + off, sz)], sem_z))
                off += sz
            bw = bounds_v[pl.ds(slot, 16)]
            start = _lane(bw, 0, iota)
            end = _lane(bw, 1, iota)
            wbase = pl.multiple_of(start & ~7, 8)
            npair = (end - wbase + 255) // 256

            def mask_fix(b, woff):
                for j in range(8):
                    pos = woff + j * 16 + iota
                    li = li_v[b, pl.ds(j * 16, 16)]
                    valid = (pos >= start) & (pos < end)
                    li_v[b, pl.ds(j * 16, 16)] = (
                        jnp.where(valid, li, r) + base_row)

            i0, l0 = idx_issue(0, wbase)
            i0.wait()
            l0.wait()
            gather_issue(0)
            for d in zd:
                d.wait()

            def pair(j, carry):
                woff0 = pl.multiple_of(wbase + j * 256, 8)
                woff1 = woff0 + 128
                woff2 = woff0 + 256
                i1, l1 = idx_issue(1, woff1)
                mask_fix(0, woff0)
                gather_wait(0)
                i1.wait()
                l1.wait()
                g1 = gather_issue(1)
                scatter(0)
                i2, l2 = idx_issue(0, woff2)
                mask_fix(1, woff1)
                g1.wait()
                i2.wait()
                l2.wait()
                gather_issue(0)
                scatter(1)
                return carry
            lax.fori_loop(0, npair, pair, 0)
            gather_wait(0)
            pltpu.sync_copy(acc.at[pl.ds(base_row, r)],
                            out_hbm.at[pl.ds(slot * r, r)])

    return seg


_SEG_LC = _make_segsum(NL, 7, 720, 256)
_SEG_CL = _make_segsum(NCL, 5, 720, 192)


def _sorted_cells(dst, src, r, nslots, nbounds_pad):
    dst_s, src_s = jax.lax.sort((dst, src), num_keys=1)
    loc = (dst_s % r).astype(jnp.int32)
    bounds = jnp.searchsorted(
        dst_s, jnp.arange(nslots + 1, dtype=jnp.int32) * r).astype(jnp.int32)
    bounds = jnp.concatenate(
        [bounds, jnp.full((nbounds_pad - nslots - 1,), NE, jnp.int32)])
    src_pad = jnp.concatenate([src_s, jnp.zeros((NE_PAD - NE,), jnp.int32)])
    loc_pad = jnp.concatenate([loc, jnp.full((NE_PAD - NE,), r, jnp.int32)])
    return src_pad, loc_pad, bounds


def kernel(lit_idx, clause_idx, n_vars, n_lits, n_clauses,
           L_init_w, L_init_b, C_init_w, C_init_b,
           Lmsg_W1, Lmsg_b1, Lmsg_W2, Lmsg_b2,
           Cmsg_W1, Cmsg_b1, Cmsg_W2, Cmsg_b2,
           Lu_Wih, Lu_Whh, Lu_bih, Lu_bhh,
           Cu_Wih, Cu_Whh, Cu_bih, Cu_bhh,
           V_W1, V_b1, V_W2, V_b2):
    lit_idx = lit_idx.astype(jnp.int32)
    clause_idx = clause_idx.astype(jnp.int32)

    lc_src, lc_loc, lc_bounds = _sorted_cells(clause_idx, lit_idx, 720, 224, 256)
    cl_src, cl_loc, cl_bounds = _sorted_cells(lit_idx, clause_idx, 720, 160, 192)

    bf = jnp.bfloat16
    lmsg_w1t = Lmsg_W1.T.astype(bf)
    lmsg_w2t = Lmsg_W2.T.astype(bf)
    cmsg_w1t = Cmsg_W1.T.astype(bf)
    cmsg_w2t = Cmsg_W2.T.astype(bf)
    lmsg_b1 = Lmsg_b1.reshape(1, DIM)
    lmsg_b2 = Lmsg_b2.reshape(1, DIM)
    cmsg_b1 = Cmsg_b1.reshape(1, DIM)
    cmsg_b2 = Cmsg_b2.reshape(1, DIM)
    cu_wih_t = Cu_Wih.T.astype(bf)
    cu_whh_t = Cu_Whh.T.astype(bf)
    cu_b = (Cu_bih + Cu_bhh).reshape(1, 4 * DIM)
    lu_wih_a_t = Lu_Wih.T[:DIM].astype(bf)
    lu_wih_b_t = Lu_Wih.T[DIM:].astype(bf)
    lu_whh_t = Lu_Whh.T.astype(bf)
    lu_b = (Lu_bih + Lu_bhh).reshape(1, 4 * DIM)
    v_w1a_t = V_W1.T[:DIM]
    v_w1b_t = V_W1.T[DIM:]
    v_b1 = V_b1.reshape(1, DIM)
    v_w2t = V_W2.T
    v_b2 = V_b2.reshape(1, 1)

    l0 = (L_init_w[:, 0] + L_init_b).reshape(1, DIM)
    c0 = (C_init_w[:, 0] + C_init_b).reshape(1, DIM)
    L_h = jnp.tile(l0, (NL, 1))
    L_c = jnp.zeros((NL, DIM), jnp.float32)
    C_h = jnp.tile(c0, (NCL, 1))
    C_c = jnp.zeros((NCL, DIM), jnp.float32)

    for _ in range(N_ROUNDS):
        L_pre = _mlp_call(L_h, lmsg_w1t, lmsg_b1, lmsg_w2t, lmsg_b2)
        LC = _SEG_LC(lc_src, lc_loc, lc_bounds, L_pre)
        C_h, C_c, C_pre = _clstm_call(LC, C_h, C_c, cu_wih_t, cu_whh_t, cu_b,
                                      cmsg_w1t, cmsg_b1, cmsg_w2t, cmsg_b2)
        CL = _SEG_CL(cl_src, cl_loc, cl_bounds, C_pre)
        L_h, L_c = _llstm_call(CL, L_h, L_c, lu_wih_a_t, lu_wih_b_t,
                               lu_whh_t, lu_b)

    return _vote_call(L_h, v_w1a_t, v_w1b_t, v_b1, v_w2t, v_b2)


# speedup vs baseline: 1.0755x; 1.0004x over previous
import functools

import jax
import jax.numpy as jnp
from jax import lax
from jax.experimental import pallas as pl
from jax.experimental.pallas import tpu as pltpu
from jax.experimental.pallas import tpu_sc as plsc

DIM = 128
NV = 50000
NL = 100000
NCL = 150000
NE = 600000
NE_PAD = NE + 512
B = 2000

N_ROUNDS = 4
NSC = 2
NSUB = 16


def _mlp_math(x, w1t, b1, w2t, b2):
    h = jnp.maximum(jnp.dot(x, w1t, preferred_element_type=jnp.float32) + b1, 0.0)
    return jnp.dot(h, w2t, preferred_element_type=jnp.float32) + b2


def _mlp_body(x_ref, w1t_ref, b1_ref, w2t_ref, b2_ref, o_ref):
    o_ref[...] = _mlp_math(x_ref[...], w1t_ref[...], b1_ref[...],
                           w2t_ref[...], b2_ref[...])


def _mlp_call(x, w1t, b1, w2t, b2):
    n = x.shape[0]
    assert n % B == 0
    return pl.pallas_call(
        _mlp_body,
        grid=(n // B,),
        in_specs=[
            pl.BlockSpec((B, DIM), lambda i: (i, 0)),
            pl.BlockSpec((DIM, DIM), lambda i: (0, 0)),
            pl.BlockSpec((1, DIM), lambda i: (0, 0)),
            pl.BlockSpec((DIM, DIM), lambda i: (0, 0)),
            pl.BlockSpec((1, DIM), lambda i: (0, 0)),
        ],
        out_specs=pl.BlockSpec((B, DIM), lambda i: (i, 0)),
        out_shape=jax.ShapeDtypeStruct((n, DIM), jnp.float32),
    )(x, w1t, b1, w2t, b2)


def _lstm_gates(g, c):
    i = g[:, :DIM]
    f = g[:, DIM:2 * DIM]
    gg = g[:, 2 * DIM:3 * DIM]
    o = g[:, 3 * DIM:]
    c_new = jax.nn.sigmoid(f) * c + jax.nn.sigmoid(i) * jnp.tanh(gg)
    h_new = jax.nn.sigmoid(o) * jnp.tanh(c_new)
    return h_new, c_new


def _clstm_body(x_ref, h_ref, c_ref, wih_ref, whh_ref, b_ref,
                w1t_ref, b1_ref, w2t_ref, b2_ref,
                h_out, c_out, pre_out):
    g = (jnp.dot(x_ref[...], wih_ref[...], preferred_element_type=jnp.float32)
         + jnp.dot(h_ref[...], whh_ref[...], preferred_element_type=jnp.float32)
         + b_ref[...])
    h_new, c_new = _lstm_gates(g, c_ref[...])
    h_out[...] = h_new
    c_out[...] = c_new
    pre_out[...] = _mlp_math(h_new, w1t_ref[...], b1_ref[...],
                             w2t_ref[...], b2_ref[...])


def _clstm_call(x_pad, h, c, wih_t, whh_t, b, w1t, b1, w2t, b2):
    n = h.shape[0]
    assert n % B == 0
    row = pl.BlockSpec((B, DIM), lambda i: (i, 0))
    full = lambda r, cdim: pl.BlockSpec((r, cdim), lambda i: (0, 0))
    return pl.pallas_call(
        _clstm_body,
        grid=(n // B,),
        in_specs=[
            row, row, row,
            full(DIM, 4 * DIM), full(DIM, 4 * DIM), full(1, 4 * DIM),
            full(DIM, DIM), full(1, DIM), full(DIM, DIM), full(1, DIM),
        ],
        out_specs=[row, row, row],
        out_shape=[
            jax.ShapeDtypeStruct((n, DIM), jnp.float32),
            jax.ShapeDtypeStruct((n, DIM), jnp.float32),
            jax.ShapeDtypeStruct((n, DIM), jnp.float32),
        ],
    )(x_pad, h, c, wih_t, whh_t, b, w1t, b1, w2t, b2)


def _llstm_body(cl_ref, flip_ref, h_ref, c_ref,
                wih_a_ref, wih_b_ref, whh_ref, b_ref,
                h_out, c_out):
    g = (jnp.dot(cl_ref[...], wih_a_ref[...], preferred_element_type=jnp.float32)
         + jnp.dot(flip_ref[...], wih_b_ref[...], preferred_element_type=jnp.float32)
         + jnp.dot(h_ref[...], whh_ref[...], preferred_element_type=jnp.float32)
         + b_ref[...])
    h_new, c_new = _lstm_gates(g, c_ref[...])
    h_out[...] = h_new
    c_out[...] = c_new


def _llstm_call(cl_pad, h, c, wih_a_t, wih_b_t, whh_t, b):
    n = h.shape[0]
    assert n % B == 0
    nb = n // B
    half = nb // 2
    row = pl.BlockSpec((B, DIM), lambda i: (i, 0))
    flip_spec = pl.BlockSpec((B, DIM), lambda i: ((i + half) % nb, 0))
    full = lambda r, cdim: pl.BlockSpec((r, cdim), lambda i: (0, 0))
    return pl.pallas_call(
        _llstm_body,
        grid=(nb,),
        in_specs=[
            row, flip_spec, row, row,
            full(DIM, 4 * DIM), full(DIM, 4 * DIM), full(DIM, 4 * DIM),
            full(1, 4 * DIM),
        ],
        out_specs=[row, row],
        out_shape=[
            jax.ShapeDtypeStruct((n, DIM), jnp.float32),
            jax.ShapeDtypeStruct((n, DIM), jnp.float32),
        ],
    )(cl_pad, h, h, c, wih_a_t, wih_b_t, whh_t, b)


def _vote_body(a_ref, b2h_ref, w1a_ref, w1b_ref, b1_ref, w2t_ref, b2_ref, o_ref):
    h = jnp.maximum(
        jnp.dot(a_ref[...], w1a_ref[...], preferred_element_type=jnp.float32)
        + jnp.dot(b2h_ref[...], w1b_ref[...], preferred_element_type=jnp.float32)
        + b1_ref[...], 0.0)
    o_ref[...] = jnp.dot(h, w2t_ref[...], preferred_element_type=jnp.float32) + b2_ref[...]


def _vote_call(l_h, w1a_t, w1b_t, b1, w2t, b2):
    nb = NV // B
    half = NL // B // 2
    row = pl.BlockSpec((B, DIM), lambda i: (i, 0))
    row_b = pl.BlockSpec((B, DIM), lambda i: (i + half, 0))
    full = lambda r, cdim: pl.BlockSpec((r, cdim), lambda i: (0, 0))
    return pl.pallas_call(
        _vote_body,
        grid=(nb,),
        in_specs=[
            row, row_b,
            full(DIM, DIM), full(DIM, DIM), full(1, DIM),
            full(DIM, 1), full(1, 1),
        ],
        out_specs=pl.BlockSpec((B, 1), lambda i: (i, 0)),
        out_shape=jax.ShapeDtypeStruct((NV, 1), jnp.float32),
    )(l_h, l_h, w1a_t, w1b_t, b1, w2t, b2)


def _lane(v, i, iota):
    return jnp.sum(jnp.where(iota == i, v, 0), axis=0)


def _make_segsum(n_in, n_pass, r, nbounds_pad):
    assert r % 8 == 0
    stride = r + 8
    n_out_pad = n_pass * NSC * NSUB * r
    mesh = plsc.VectorSubcoreMesh(core_axis_name="c", subcore_axis_name="s")

    @functools.partial(
        pl.kernel,
        out_type=jax.ShapeDtypeStruct((n_out_pad, DIM), jnp.float32),
        mesh=mesh,
        compiler_params=pltpu.CompilerParams(needs_layout_passes=False),
        scratch_types=[
            pltpu.VMEM((nbounds_pad,), jnp.int32),
            pltpu.VMEM((2, 128), jnp.int32),
            pltpu.VMEM((2, 128), jnp.int32),
            pltpu.VMEM((2, 128, DIM), jnp.float32),
            pltpu.VMEM((32, DIM), jnp.float32),
            pltpu.VMEM_SHARED((NSUB * stride, DIM), jnp.float32),
            pltpu.SemaphoreType.DMA((2,)),
            pltpu.SemaphoreType.DMA((2,)),
            pltpu.SemaphoreType.DMA((2,)),
            pltpu.SemaphoreType.DMA,
        ],
    )
    def seg(g_hbm, l_hbm, b_hbm, x_hbm, out_hbm,
            bounds_v, gi_v, li_v, rows_v, zeros_v, acc,
            sem_g, sem_i, sem_l, sem_z):
        c = lax.axis_index("c")
        s = lax.axis_index("s")
        iota = lax.iota(jnp.int32, 16)
        pltpu.sync_copy(b_hbm, bounds_v)

        def zloop(i, carry):
            for j in range(8):
                zeros_v[i, pl.ds(j * 16, 16)] = jnp.zeros((16,), jnp.float32)
            return carry
        lax.fori_loop(0, 32, zloop, 0)

        base_row = s * stride

        def idx_issue(b, woff):
            return (pltpu.async_copy(g_hbm.at[pl.ds(woff, 128)],
                                     gi_v.at[b], sem_i.at[b]),
                    pltpu.async_copy(l_hbm.at[pl.ds(woff, 128)],
                                     li_v.at[b], sem_l.at[b]))

        def gather_issue(b):
            return pltpu.async_copy(x_hbm.at[gi_v.at[b]], rows_v.at[b],
                                    sem_g.at[b])

        def gather_wait(b):
            pltpu.make_async_copy(x_hbm.at[gi_v.at[b]], rows_v.at[b],
                                  sem_g.at[b]).wait()

        def scatter(b):
            pltpu.sync_copy(rows_v.at[b], acc.at[li_v.at[b]], add=True)

        for p in range(n_pass):
            slot = (p * NSC + c) * NSUB + s
            zd = []
            off = 0
            while off < stride:
                sz = min(32, stride - off)
                zd.append(pltpu.async_copy(
                    zeros_v.at[pl.ds(0, sz)],
                    acc.at[pl.ds(base_row + off, sz)], sem_z))
                off += sz
            bw = bounds_v[pl.ds(slot, 16)]
            start = _lane(bw, 0, iota)
            end = _lane(bw, 1, iota)
            wbase = pl.multiple_of(start & ~7, 8)
            npair = (end - wbase + 255) // 256

            def mask_fix(b, woff):
                for j in range(8):
                    pos = woff + j * 16 + iota
                    li = li_v[b, pl.ds(j * 16, 16)]
                    valid = (pos >= start) & (pos < end)
                    li_v[b, pl.ds(j * 16, 16)] = (
                        jnp.where(valid, li, r) + base_row)

            i0, l0 = idx_issue(0, wbase)
            i0.wait()
            l0.wait()
            gather_issue(0)
            for d in zd:
                d.wait()

            def pair(j, carry):
                woff0 = pl.multiple_of(wbase + j * 256, 8)
                woff1 = woff0 + 128
                woff2 = woff0 + 256
                i1, l1 = idx_issue(1, woff1)
                mask_fix(0, woff0)
                gather_wait(0)
                i1.wait()
                l1.wait()
                g1 = gather_issue(1)
                scatter(0)
                i2, l2 = idx_issue(0, woff2)
                mask_fix(1, woff1)
                g1.wait()
                i2.wait()
                l2.wait()
                gather_issue(0)
                scatter(1)
                return carry
            lax.fori_loop(0, npair, pair, 0)
            gather_wait(0)
            pltpu.sync_copy(acc.at[pl.ds(base_row, r)],
                            out_hbm.at[pl.ds(slot * r, r)])

    return seg


_SEG_LC = _make_segsum(NL, 7, 720, 256)
_SEG_CL = _make_segsum(NCL, 5, 720, 192)


def _sorted_cells(dst, src, r, nslots, nbounds_pad):
    dst_s, src_s = jax.lax.sort((dst, src), num_keys=1)
    loc = (dst_s % r).astype(jnp.int32)
    bounds = jnp.searchsorted(
        dst_s, jnp.arange(nslots + 1, dtype=jnp.int32) * r).astype(jnp.int32)
    bounds = jnp.concatenate(
        [bounds, jnp.full((nbounds_pad - nslots - 1,), NE, jnp.int32)])
    src_pad = jnp.concatenate([src_s, jnp.zeros((NE_PAD - NE,), jnp.int32)])
    loc_pad = jnp.concatenate([loc, jnp.full((NE_PAD - NE,), r, jnp.int32)])
    return src_pad, loc_pad, bounds


def kernel(lit_idx, clause_idx, n_vars, n_lits, n_clauses,
           L_init_w, L_init_b, C_init_w, C_init_b,
           Lmsg_W1, Lmsg_b1, Lmsg_W2, Lmsg_b2,
           Cmsg_W1, Cmsg_b1, Cmsg_W2, Cmsg_b2,
           Lu_Wih, Lu_Whh, Lu_bih, Lu_bhh,
           Cu_Wih, Cu_Whh, Cu_bih, Cu_bhh,
           V_W1, V_b1, V_W2, V_b2):
    lit_idx = lit_idx.astype(jnp.int32)
    clause_idx = clause_idx.astype(jnp.int32)

    lc_src, lc_loc, lc_bounds = _sorted_cells(clause_idx, lit_idx, 720, 224, 256)
    cl_src, cl_loc, cl_bounds = _sorted_cells(lit_idx, clause_idx, 720, 160, 192)

    lmsg_w1t = Lmsg_W1.T
    lmsg_w2t = Lmsg_W2.T
    cmsg_w1t = Cmsg_W1.T
    cmsg_w2t = Cmsg_W2.T
    lmsg_b1 = Lmsg_b1.reshape(1, DIM)
    lmsg_b2 = Lmsg_b2.reshape(1, DIM)
    cmsg_b1 = Cmsg_b1.reshape(1, DIM)
    cmsg_b2 = Cmsg_b2.reshape(1, DIM)
    cu_wih_t = Cu_Wih.T
    cu_whh_t = Cu_Whh.T
    cu_b = (Cu_bih + Cu_bhh).reshape(1, 4 * DIM)
    lu_wih_a_t = Lu_Wih.T[:DIM]
    lu_wih_b_t = Lu_Wih.T[DIM:]
    lu_whh_t = Lu_Whh.T
    lu_b = (Lu_bih + Lu_bhh).reshape(1, 4 * DIM)
    v_w1a_t = V_W1.T[:DIM]
    v_w1b_t = V_W1.T[DIM:]
    v_b1 = V_b1.reshape(1, DIM)
    v_w2t = V_W2.T
    v_b2 = V_b2.reshape(1, 1)

    l0 = (L_init_w[:, 0] + L_init_b).reshape(1, DIM)
    c0 = (C_init_w[:, 0] + C_init_b).reshape(1, DIM)
    L_h = jnp.tile(l0, (NL, 1))
    L_c = jnp.zeros((NL, DIM), jnp.float32)
    C_h = jnp.tile(c0, (NCL, 1))
    C_c = jnp.zeros((NCL, DIM), jnp.float32)

    for _ in range(N_ROUNDS):
        L_pre = _mlp_call(L_h, lmsg_w1t, lmsg_b1, lmsg_w2t, lmsg_b2)
        LC = _SEG_LC(lc_src, lc_loc, lc_bounds, L_pre)
        C_h, C_c, C_pre = _clstm_call(LC, C_h, C_c, cu_wih_t, cu_whh_t, cu_b,
                                      cmsg_w1t, cmsg_b1, cmsg_w2t, cmsg_b2)
        CL = _SEG_CL(cl_src, cl_loc, cl_bounds, C_pre)
        L_h, L_c = _llstm_call(CL, L_h, L_c, lu_wih_a_t, lu_wih_b_t,
                               lu_whh_t, lu_b)

    return _vote_call(L_h, v_w1a_t, v_w1b_t, v_b1, v_w2t, v_b2)
